# baseline (speedup 1.0000x reference)
import jax
import jax.numpy as jnp
from jax.experimental import pallas as pl
from jax.experimental.pallas import tpu as pltpu

_K = 16
_BIG = 0x7FFFFFFF


def _extract16(vals, idxs, exact_ties):
    r = vals.shape[0]
    ok = jax.lax.broadcasted_iota(jnp.int32, (r, _K), 1)

    def body(p, carry):
        vals, oidx, od = carry
        m = jnp.min(vals, axis=1)
        eq = vals == m[:, None]
        j = jnp.min(jnp.where(eq, idxs, jnp.int32(_BIG)), axis=1)
        oidx = jnp.where(ok == p, j[:, None], oidx)
        od = jnp.where(ok == p, m[:, None], od)
        if exact_ties:
            vals = jnp.where(eq & (idxs == j[:, None]), jnp.inf, vals)
        else:
            vals = jnp.where(eq, jnp.inf, vals)
        return vals, oidx, od

    _, oidx, od = jax.lax.fori_loop(
        0, _K, body,
        (vals, jnp.zeros((r, _K), jnp.int32), jnp.zeros((r, _K), jnp.float32)),
    )
    return oidx, od


def _knn_block_kernel(q_ref, k_ref, idx_ref, d_ref):
    q = q_ref[...]
    ks = k_ref[...]
    sqk = jnp.sum(ks * ks, axis=1)
    sqq = jnp.sum(q * q, axis=1)
    dot = jax.lax.dot_general(
        q, ks, (((1,), (1,)), ((), ())),
        preferred_element_type=jnp.float32,
        precision=jax.lax.Precision.DEFAULT,
    )
    d2 = jnp.maximum(sqq[:, None] - 2.0 * dot + sqk[None, :], 0.0)

    r, w = d2.shape
    g = w // 16

    d3 = d2.reshape(r, 16, g)
    i3 = jax.lax.broadcasted_iota(jnp.int32, (r, 16, g), 1) * g \
        + jax.lax.broadcasted_iota(jnp.int32, (r, 16, g), 2)
    m1 = jnp.min(d3, axis=1)
    eq1 = d3 == m1[:, None, :]
    j1 = jnp.min(jnp.where(eq1, i3, jnp.int32(_BIG)), axis=1)
    d3m = jnp.where(eq1, jnp.inf, d3)
    m2 = jnp.min(d3m, axis=1)
    eq2 = d3m == m2[:, None, :]
    j2 = jnp.min(jnp.where(eq2, i3, jnp.int32(_BIG)), axis=1)
    cvals = jnp.concatenate([m1, m2], axis=1)
    cidx = jnp.concatenate([j1, j2], axis=1)

    m3 = jnp.min(jnp.where(eq2, jnp.inf, d3m), axis=1)

    oidx, od = _extract16(cvals, cidx, exact_ties=True)

    s = jnp.max(od, axis=1)[:, None]
    bad = jnp.any(m3 <= s)

    ii = jax.lax.broadcasted_iota(jnp.int32, (r, w), 1)
    oidx, od = jax.lax.cond(
        bad, lambda: _extract16(d2, ii, exact_ties=True),
        lambda: (oidx, od))

    idx_ref[...] = oidx
    d_ref[...] = od


def kernel(embeds):
    n, d = embeds.shape
    r = 128
    grid = (n // r,)
    nbr_idx, knn_dists = pl.pallas_call(
        _knn_block_kernel,
        grid=grid,
        compiler_params=pltpu.CompilerParams(
            dimension_semantics=("parallel",)),
        in_specs=[
            pl.BlockSpec((r, d), lambda i: (i, 0)),
            pl.BlockSpec((n, d), lambda i: (0, 0)),
        ],
        out_specs=[
            pl.BlockSpec((r, _K), lambda i: (i, 0)),
            pl.BlockSpec((r, _K), lambda i: (i, 0)),
        ],
        out_shape=[
            jax.ShapeDtypeStruct((n, _K), jnp.int32),
            jax.ShapeDtypeStruct((n, _K), jnp.float32),
        ],
    )(embeds, embeds)
    row = nbr_idx.reshape(-1)
    col = jnp.repeat(jnp.arange(n, dtype=row.dtype), _K)
    edge_index = jnp.stack([row, col], axis=0)
    return edge_index, knn_dists


# speedup vs baseline: 1.4653x; 1.4653x over previous
import jax
import jax.numpy as jnp
from jax.experimental import pallas as pl
from jax.experimental.pallas import tpu as pltpu

_K = 16
_BIG = 0x7FFFFFFF


def _extract16(vals, idxs, exact_ties):
    r = vals.shape[0]
    ok = jax.lax.broadcasted_iota(jnp.int32, (r, _K), 1)

    def body(p, carry):
        vals, oidx, od = carry
        m = jnp.min(vals, axis=1)
        eq = vals == m[:, None]
        j = jnp.min(jnp.where(eq, idxs, jnp.int32(_BIG)), axis=1)
        oidx = jnp.where(ok == p, j[:, None], oidx)
        od = jnp.where(ok == p, m[:, None], od)
        if exact_ties:
            vals = jnp.where(eq & (idxs == j[:, None]), jnp.inf, vals)
        else:
            vals = jnp.where(eq, jnp.inf, vals)
        return vals, oidx, od

    _, oidx, od = jax.lax.fori_loop(
        0, _K, body,
        (vals, jnp.zeros((r, _K), jnp.int32), jnp.zeros((r, _K), jnp.float32)),
    )
    return oidx, od


def _knn_block_kernel(q_ref, k_ref, idx_ref, d_ref):
    q = q_ref[...]
    ks = k_ref[...]
    sqk = jnp.sum(ks * ks, axis=1)
    sqq = jnp.sum(q * q, axis=1)
    dot = jax.lax.dot_general(
        q, ks, (((1,), (1,)), ((), ())),
        preferred_element_type=jnp.float32,
        precision=jax.lax.Precision.DEFAULT,
    )

    r = q.shape[0]
    w = dot.shape[1]
    g = w // 16
    sqqc = sqq[:, None]

    def member(j):
        return jnp.maximum(
            sqqc - 2.0 * dot[:, j * g:(j + 1) * g]
            + sqk[None, j * g:(j + 1) * g], 0.0)

    lo = member(0)
    lo_j = jnp.zeros((r, g), jnp.int32)
    hi = jnp.full((r, g), jnp.inf, jnp.float32)
    hi_j = jnp.zeros((r, g), jnp.int32)
    m3 = jnp.full((r, g), jnp.inf, jnp.float32)
    for j in range(1, 16):
        v = member(j)
        jj = jnp.int32(j)
        isl = v < lo
        dem = jnp.where(isl, lo, v)
        dem_j = jnp.where(isl, lo_j, jj)
        lo = jnp.where(isl, v, lo)
        lo_j = jnp.where(isl, jj, lo_j)
        ish = dem < hi
        loser = jnp.where(ish, hi, dem)
        hi = jnp.where(ish, dem, hi)
        hi_j = jnp.where(ish, dem_j, hi_j)
        m3 = jnp.minimum(m3, loser)

    bio = jax.lax.broadcasted_iota(jnp.int32, (r, g), 1)
    cvals = jnp.concatenate([lo, hi], axis=1)
    cidx = jnp.concatenate([lo_j * g + bio, hi_j * g + bio], axis=1)

    oidx, od = _extract16(cvals, cidx, exact_ties=True)

    s = jnp.max(od, axis=1)[:, None]
    bad = jnp.any(m3 <= s)

    def fallback():
        d2 = jnp.maximum(sqqc - 2.0 * dot + sqk[None, :], 0.0)
        ii = jax.lax.broadcasted_iota(jnp.int32, (r, w), 1)
        return _extract16(d2, ii, exact_ties=True)

    oidx, od = jax.lax.cond(bad, fallback, lambda: (oidx, od))

    idx_ref[...] = oidx
    d_ref[...] = od


def kernel(embeds):
    n, d = embeds.shape
    r = 128
    grid = (n // r,)
    nbr_idx, knn_dists = pl.pallas_call(
        _knn_block_kernel,
        grid=grid,
        compiler_params=pltpu.CompilerParams(
            dimension_semantics=("parallel",)),
        in_specs=[
            pl.BlockSpec((r, d), lambda i: (i, 0)),
            pl.BlockSpec((n, d), lambda i: (0, 0)),
        ],
        out_specs=[
            pl.BlockSpec((r, _K), lambda i: (i, 0)),
            pl.BlockSpec((r, _K), lambda i: (i, 0)),
        ],
        out_shape=[
            jax.ShapeDtypeStruct((n, _K), jnp.int32),
            jax.ShapeDtypeStruct((n, _K), jnp.float32),
        ],
    )(embeds, embeds)
    row = nbr_idx.reshape(-1)
    col = jnp.repeat(jnp.arange(n, dtype=row.dtype), _K)
    edge_index = jnp.stack([row, col], axis=0)
    return edge_index, knn_dists


# speedup vs baseline: 1.5013x; 1.0246x over previous
import jax
import jax.numpy as jnp
from jax.experimental import pallas as pl
from jax.experimental.pallas import tpu as pltpu

_K = 16
_BIG = 0x7FFFFFFF


def _extract16(vals, idxs, exact_ties):
    r = vals.shape[0]
    ok = jax.lax.broadcasted_iota(jnp.int32, (r, _K), 1)

    def body(p, carry):
        vals, oidx, od = carry
        m = jnp.min(vals, axis=1)
        eq = vals == m[:, None]
        j = jnp.min(jnp.where(eq, idxs, jnp.int32(_BIG)), axis=1)
        oidx = jnp.where(ok == p, j[:, None], oidx)
        od = jnp.where(ok == p, m[:, None], od)
        if exact_ties:
            vals = jnp.where(eq & (idxs == j[:, None]), jnp.inf, vals)
        else:
            vals = jnp.where(eq, jnp.inf, vals)
        return vals, oidx, od

    _, oidx, od = jax.lax.fori_loop(
        0, _K, body,
        (vals, jnp.zeros((r, _K), jnp.int32), jnp.zeros((r, _K), jnp.float32)),
    )
    return oidx, od


def _knn_block_kernel(q_ref, k_ref, idx_ref, d_ref, sqk_ref):
    q = q_ref[...]
    ks = k_ref[...]

    @pl.when(pl.program_id(0) == 0)
    def _():
        sqk_ref[...] = jnp.sum(ks * ks, axis=1)[None, :]

    sqk = sqk_ref[...][0]
    sqq = jnp.sum(q * q, axis=1)
    dot = jax.lax.dot_general(
        q, ks, (((1,), (1,)), ((), ())),
        preferred_element_type=jnp.float32,
        precision=jax.lax.Precision.DEFAULT,
    )

    r = q.shape[0]
    w = dot.shape[1]
    g = w // 16
    sqqc = sqq[:, None]

    def member(j):
        return jnp.maximum(
            sqqc - 2.0 * dot[:, j * g:(j + 1) * g]
            + sqk[None, j * g:(j + 1) * g], 0.0)

    lo = member(0)
    lo_j = jnp.zeros((r, g), jnp.int32)
    hi = jnp.full((r, g), jnp.inf, jnp.float32)
    hi_j = jnp.zeros((r, g), jnp.int32)
    m3 = jnp.full((r, g), jnp.inf, jnp.float32)
    for j in range(1, 16):
        v = member(j)
        jj = jnp.int32(j)
        isl = v < lo
        dem = jnp.where(isl, lo, v)
        dem_j = jnp.where(isl, lo_j, jj)
        lo = jnp.where(isl, v, lo)
        lo_j = jnp.where(isl, jj, lo_j)
        ish = dem < hi
        loser = jnp.where(ish, hi, dem)
        hi = jnp.where(ish, dem, hi)
        hi_j = jnp.where(ish, dem_j, hi_j)
        m3 = jnp.minimum(m3, loser)

    bio = jax.lax.broadcasted_iota(jnp.int32, (r, g), 1)
    cvals = jnp.concatenate([lo, hi], axis=1)
    cidx = jnp.concatenate([lo_j * g + bio, hi_j * g + bio], axis=1)

    oidx, od = _extract16(cvals, cidx, exact_ties=True)

    s = jnp.max(od, axis=1)[:, None]
    bad = jnp.any(m3 <= s)

    def fallback():
        d2 = jnp.maximum(sqqc - 2.0 * dot + sqk[None, :], 0.0)
        ii = jax.lax.broadcasted_iota(jnp.int32, (r, w), 1)
        return _extract16(d2, ii, exact_ties=True)

    oidx, od = jax.lax.cond(bad, fallback, lambda: (oidx, od))

    idx_ref[...] = oidx
    d_ref[...] = od


def kernel(embeds):
    n, d = embeds.shape
    r = 128
    grid = (n // r,)
    nbr_idx, knn_dists = pl.pallas_call(
        _knn_block_kernel,
        grid=grid,
        compiler_params=pltpu.CompilerParams(
            dimension_semantics=("arbitrary",)),
        scratch_shapes=[pltpu.VMEM((1, n), jnp.float32)],
        in_specs=[
            pl.BlockSpec((r, d), lambda i: (i, 0)),
            pl.BlockSpec((n, d), lambda i: (0, 0)),
        ],
        out_specs=[
            pl.BlockSpec((r, _K), lambda i: (i, 0)),
            pl.BlockSpec((r, _K), lambda i: (i, 0)),
        ],
        out_shape=[
            jax.ShapeDtypeStruct((n, _K), jnp.int32),
            jax.ShapeDtypeStruct((n, _K), jnp.float32),
        ],
    )(embeds, embeds)
    row = nbr_idx.reshape(-1)
    col = jnp.repeat(jnp.arange(n, dtype=row.dtype), _K)
    edge_index = jnp.stack([row, col], axis=0)
    return edge_index, knn_dists
